# baseline (speedup 1.0000x reference)
import functools
import math

import jax
import jax.numpy as jnp
import numpy as np
from jax.experimental import pallas as pl
from jax.experimental.pallas import tpu as pltpu
from jax.experimental.pallas import tpu_sc as plsc

_RADIUS = 1.0
_PCTS = (0.02, 0.04, 0.06, 0.08, 0.1)
_B, _N, _C = 4, 2048, 3
_S = int(_N * 0.05)
_SP = 128
_GRP = 6
_NW = 32
_GW = (_B * _S + _NW - 1) // _NW
_BIG = np.float32(1e30)
_E = np.float32(0.1)


def _cfg():
    cfgs = []
    for p in _PCTS:
        ns = int(_N * p)
        kp = ((ns + 7) // 8) * 8
        r = math.sqrt(p * _RADIUS)
        r2 = np.float32(r * r)
        disk_area = math.pi * _RADIUS ** 2 / _N
        el = np.float32(math.sqrt(disk_area))
        scale = np.float32((p * 100.0) ** 2)
        cfgs.append((ns, kp, r2, el, scale))
    return tuple(cfgs)


_CFGS = _cfg()


def _fps_kernel(pt_ref, nxyz_ref):
    pt = pt_ref[...]
    lane = jax.lax.broadcasted_iota(jnp.int32, (1, _N), 1)
    lane_s = jax.lax.broadcasted_iota(jnp.int32, (1, 1, _SP), 2)

    def body(i, state):
        dists, far, cacc = state
        oh = lane == far
        cent = jnp.sum(jnp.where(oh[:, None, :], pt, 0.0), axis=2,
                       keepdims=True)
        cacc = jnp.where(lane_s == i, cent, cacc)
        d = jnp.sum((pt - cent) ** 2, axis=1)
        dists = jnp.minimum(dists, d)
        maxv = jnp.max(dists, axis=1, keepdims=True)
        far = jnp.min(jnp.where(dists == maxv, lane, _N), axis=1,
                      keepdims=True)
        return dists, far, cacc

    d0 = jnp.full((_B, _N), 1e10, jnp.float32)
    f0 = jnp.zeros((_B, 1), jnp.int32)
    c0 = jnp.zeros((_B, 3, _SP), jnp.float32)
    _, _, cacc = jax.lax.fori_loop(0, _S, body, (d0, f0, c0))
    nxyz_ref[...] = cacc


def _sc_gather_body(pt_hbm, nxyz_hbm, rows_hbm, cnt_hbm,
                    pt_v, nx_v, row_v, cnt_v):
    wid = jax.lax.axis_index("s") * 2 + jax.lax.axis_index("c")
    il = jax.lax.iota(jnp.int32, 16)

    for i in range(_GW):
        g = wid * _GW + i

        @pl.when(g < _B * _S)
        def _():
            b = g // _S
            s = g - b * _S
            pltpu.sync_copy(pt_hbm.at[b], pt_v)
            pltpu.sync_copy(nxyz_hbm.at[b], nx_v)
            ch = (s // 16) * 16
            lsplat = jnp.full((16, 1), s - ch, jnp.int32)
            gdn = jax.lax.GatherDimensionNumbers(
                offset_dims=(), collapsed_slice_dims=(0,),
                start_index_map=(0,))
            pib = jax.lax.GatherScatterMode.PROMISE_IN_BOUNDS

            def splat(v):
                return jax.lax.gather(v, lsplat, gdn, slice_sizes=(1,),
                                      mode=pib)
            cx = splat(nx_v[0, pl.ds(ch, 16)])
            cy = splat(nx_v[1, pl.ds(ch, 16)])
            cz = splat(nx_v[2, pl.ds(ch, 16)])

            def chunk(j, bases):
                x = pt_v[0, pl.ds(j * 16, 16)]
                y = pt_v[1, pl.ds(j * 16, 16)]
                z = pt_v[2, pl.ds(j * 16, 16)]
                dx, dy, dz = x - cx, y - cy, z - cz
                d2 = dx * dx + dy * dy + dz * dz
                sqv = x * x + y * y + z * z
                out = []
                for pi, (ns, kp, r2, el, scale) in enumerate(_CFGS):
                    m = d2 <= r2
                    rk = jnp.where(m, 1, 0)
                    for k in (1, 2, 4, 8):
                        sh = jax.lax.gather(
                            rk, jnp.maximum(il - k, 0)[:, None], gdn,
                            slice_sizes=(1,), mode=pib)
                        rk = rk + jnp.where(il >= k, sh, 0)
                    pos = bases[pi] + rk - 1
                    wm = m & (pos < ns)
                    r0 = 4 * pi * 256
                    p0 = jnp.where(wm, pos + r0, 20 * 256)
                    plsc.store_scatter(row_v, [p0], x)
                    plsc.store_scatter(row_v, [jnp.where(wm, pos + (r0 + 256),
                                                         20 * 256)], y)
                    plsc.store_scatter(row_v, [jnp.where(wm, pos + (r0 + 512),
                                                         20 * 256)], z)
                    plsc.store_scatter(row_v, [jnp.where(wm, pos + (r0 + 768),
                                                         20 * 256)], sqv)
                    out.append(bases[pi]
                               + plsc.all_reduce_population_count(m))
                return tuple(out)

            z16 = jnp.zeros((16,), jnp.int32)
            bases = jax.lax.fori_loop(0, _N // 16, chunk, (z16,) * 5)
            cnt = jnp.zeros((16,), jnp.int32)
            for pi in range(5):
                cnt = jnp.where(il == pi, bases[pi], cnt)
            cnt_v[...] = cnt
            pltpu.sync_copy(row_v, rows_hbm.at[b, s])
            pltpu.sync_copy(cnt_v, cnt_hbm.at[b, s])


def _sc_gather(pt, nxyz):
    mesh = plsc.VectorSubcoreMesh(core_axis_name="c", subcore_axis_name="s")
    return pl.kernel(
        _sc_gather_body,
        mesh=mesh,
        compiler_params=pltpu.CompilerParams(needs_layout_passes=False,
                                             use_tc_tiling_on_sc=False),
        out_type=[
            jax.ShapeDtypeStruct((_B, _S, 24 * 256), jnp.float32),
            jax.ShapeDtypeStruct((_B, _S, 16), jnp.int32),
        ],
        scratch_types=[
            pltpu.VMEM((3, _N), jnp.float32),
            pltpu.VMEM((3, _SP), jnp.float32),
            pltpu.VMEM((24 * 256,), jnp.float32),
            pltpu.VMEM((16,), jnp.int32),
        ],
    )(pt, nxyz)


def _center_loss(rows_g, cnts):
    acc = jnp.zeros((1, 1), jnp.float32)
    dn_row = (((0,), (0,)), ((), ()))
    for i, (ns, kp, r2, el, scale) in enumerate(_CFGS):
        xr = rows_g[4 * i + 0:4 * i + 1, 0:kp]
        yr = rows_g[4 * i + 1:4 * i + 2, 0:kp]
        zr = rows_g[4 * i + 2:4 * i + 3, 0:kp]
        sqr = rows_g[4 * i + 3:4 * i + 4, 0:kp]
        count = jnp.minimum(cnts[:, i:i + 1], ns)

        kv = jax.lax.broadcasted_iota(jnp.int32, (kp, 1), 0)
        lv = jax.lax.broadcasted_iota(jnp.int32, (1, kp), 1)
        padl = lv >= count
        xb = jnp.where(padl, xr[0:1, 0:1], xr)
        yb = jnp.where(padl, yr[0:1, 0:1], yr)
        zb = jnp.where(padl, zr[0:1, 0:1], zr)
        ggr = jnp.where(padl, sqr[0:1, 0:1], sqr)
        bufT = jnp.concatenate([xb, yb, zb], axis=0).astype(jnp.bfloat16)
        G = jax.lax.dot_general(bufT, bufT, dn_row,
                                preferred_element_type=jnp.float32)
        ggc = jnp.transpose(ggr)
        d2 = jnp.maximum(ggc + ggr - 2.0 * G, 0.0)
        d2 = d2 + jnp.where(lv >= ns, _BIG, 0.0)
        min1 = jnp.min(d2, axis=1, keepdims=True)
        j0 = jnp.min(jnp.where(d2 == min1, lv, kp), axis=1,
                     keepdims=True)
        n2 = jnp.min(d2 + jnp.where(lv == j0, _BIG, 0.0), axis=1,
                     keepdims=True)
        near = jnp.sqrt(jnp.maximum(n2, 1e-12))
        contrib = jnp.where(kv < ns, near + _E, 0.0)
        u = jnp.sum(contrib, axis=0, keepdims=True) / ns
        acc = acc + (u - el) ** 2 / (el + _E) * scale
    return acc


def _group_kernel(rows_ref, cnt_ref, out_ref):
    s0 = pl.program_id(1) * _GRP
    lane_s = jax.lax.broadcasted_iota(jnp.int32, (1, _S), 1)
    row = out_ref[0]
    for g in range(_GRP):
        acc = _center_loss(rows_ref[0, g], cnt_ref[0, g])
        row = jnp.where(lane_s == s0 + g, acc, row)
    out_ref[0] = row


def kernel(pcd):
    pt = jnp.transpose(pcd, (0, 2, 1))
    nxyz = pl.pallas_call(
        _fps_kernel,
        out_shape=jax.ShapeDtypeStruct((_B, 3, _SP), jnp.float32),
    )(pt)
    rows, cnts = _sc_gather(pt, nxyz)
    rows = rows.reshape(_B, _S, 24, 256)
    cnts = cnts.reshape(_B, _S, 1, 16)
    out = pl.pallas_call(
        _group_kernel,
        grid=(_B, _S // _GRP),
        in_specs=[
            pl.BlockSpec((1, _GRP, 24, 256), lambda b, s: (b, s, 0, 0)),
            pl.BlockSpec((1, _GRP, 1, 16), lambda b, s: (b, s, 0, 0)),
        ],
        out_specs=pl.BlockSpec((1, 1, _S), lambda b, s: (b, 0, 0)),
        out_shape=jax.ShapeDtypeStruct((_B, 1, _S), jnp.float32),
        compiler_params=pltpu.CompilerParams(
            dimension_semantics=("parallel", "arbitrary")),
    )(rows, cnts)
    return jnp.mean(out) / len(_PCTS)


# speedup vs baseline: 1.1623x; 1.1623x over previous
import functools
import math

import jax
import jax.numpy as jnp
import numpy as np
from jax.experimental import pallas as pl
from jax.experimental.pallas import tpu as pltpu
from jax.experimental.pallas import tpu_sc as plsc

_RADIUS = 1.0
_PCTS = (0.02, 0.04, 0.06, 0.08, 0.1)
_B, _N, _C = 4, 2048, 3
_S = int(_N * 0.05)
_SP = 128
_GRP = 2
_NW = 32
_GW = (_B * _S + _NW - 1) // _NW
_BIG = np.float32(1e30)
_E = np.float32(0.1)


def _cfg():
    cfgs = []
    for p in _PCTS:
        ns = int(_N * p)
        kp = ((ns + 7) // 8) * 8
        r = math.sqrt(p * _RADIUS)
        r2 = np.float32(r * r)
        disk_area = math.pi * _RADIUS ** 2 / _N
        el = np.float32(math.sqrt(disk_area))
        scale = np.float32((p * 100.0) ** 2)
        cfgs.append((ns, kp, r2, el, scale))
    return tuple(cfgs)


_CFGS = _cfg()


def _fps_kernel(pt_ref, nxyz_ref):
    pt = pt_ref[...]
    lane = jax.lax.broadcasted_iota(jnp.int32, (1, _N), 1)
    lane_s = jax.lax.broadcasted_iota(jnp.int32, (1, 1, _SP), 2)

    def body(i, state):
        dists, far, cacc = state
        oh = lane == far
        cent = jnp.sum(jnp.where(oh[:, None, :], pt, 0.0), axis=2,
                       keepdims=True)
        cacc = jnp.where(lane_s == i, cent, cacc)
        d = jnp.sum((pt - cent) ** 2, axis=1)
        dists = jnp.minimum(dists, d)
        maxv = jnp.max(dists, axis=1, keepdims=True)
        far = jnp.min(jnp.where(dists == maxv, lane, _N), axis=1,
                      keepdims=True)
        return dists, far, cacc

    d0 = jnp.full((_B, _N), 1e10, jnp.float32)
    f0 = jnp.zeros((_B, 1), jnp.int32)
    c0 = jnp.zeros((_B, 3, _SP), jnp.float32)
    _, _, cacc = jax.lax.fori_loop(0, _S, body, (d0, f0, c0))
    nxyz_ref[...] = cacc


def _sc_gather_body(pt_hbm, nxyz_hbm, rows_hbm, cnt_hbm,
                    pt_v, nx_v, row_v, cnt_v):
    wid = jax.lax.axis_index("s") * 2 + jax.lax.axis_index("c")
    il = jax.lax.iota(jnp.int32, 16)

    for i in range(_GW):
        g = wid * _GW + i

        @pl.when(g < _B * _S)
        def _():
            b = g // _S
            s = g - b * _S
            pltpu.sync_copy(pt_hbm.at[b], pt_v)
            pltpu.sync_copy(nxyz_hbm.at[b], nx_v)
            ch = (s // 16) * 16
            lsplat = jnp.full((16, 1), s - ch, jnp.int32)
            gdn = jax.lax.GatherDimensionNumbers(
                offset_dims=(), collapsed_slice_dims=(0,),
                start_index_map=(0,))
            pib = jax.lax.GatherScatterMode.PROMISE_IN_BOUNDS

            def splat(v):
                return jax.lax.gather(v, lsplat, gdn, slice_sizes=(1,),
                                      mode=pib)
            cx = splat(nx_v[0, pl.ds(ch, 16)])
            cy = splat(nx_v[1, pl.ds(ch, 16)])
            cz = splat(nx_v[2, pl.ds(ch, 16)])

            def chunk(j, bases):
                x = pt_v[0, pl.ds(j * 16, 16)]
                y = pt_v[1, pl.ds(j * 16, 16)]
                z = pt_v[2, pl.ds(j * 16, 16)]
                dx, dy, dz = x - cx, y - cy, z - cz
                d2 = dx * dx + dy * dy + dz * dz
                sqv = x * x + y * y + z * z
                out = []
                for pi, (ns, kp, r2, el, scale) in enumerate(_CFGS):
                    m = d2 <= r2
                    rk = jnp.where(m, 1, 0)
                    for k in (1, 2, 4, 8):
                        sh = jax.lax.gather(
                            rk, jnp.maximum(il - k, 0)[:, None], gdn,
                            slice_sizes=(1,), mode=pib)
                        rk = rk + jnp.where(il >= k, sh, 0)
                    pos = bases[pi] + rk - 1
                    wm = m & (pos < ns)
                    r0 = 4 * pi * 256
                    p0 = jnp.where(wm, pos + r0, 20 * 256)
                    plsc.store_scatter(row_v, [p0], x)
                    plsc.store_scatter(row_v, [jnp.where(wm, pos + (r0 + 256),
                                                         20 * 256)], y)
                    plsc.store_scatter(row_v, [jnp.where(wm, pos + (r0 + 512),
                                                         20 * 256)], z)
                    plsc.store_scatter(row_v, [jnp.where(wm, pos + (r0 + 768),
                                                         20 * 256)], sqv)
                    out.append(bases[pi]
                               + plsc.all_reduce_population_count(m))
                return tuple(out)

            z16 = jnp.zeros((16,), jnp.int32)
            bases = jax.lax.fori_loop(0, _N // 16, chunk, (z16,) * 5)
            cnt = jnp.zeros((16,), jnp.int32)
            for pi in range(5):
                cnt = jnp.where(il == pi, bases[pi], cnt)
            cnt_v[...] = cnt
            pltpu.sync_copy(row_v, rows_hbm.at[b, s])
            pltpu.sync_copy(cnt_v, cnt_hbm.at[b, s])


def _sc_gather(pt, nxyz):
    mesh = plsc.VectorSubcoreMesh(core_axis_name="c", subcore_axis_name="s")
    return pl.kernel(
        _sc_gather_body,
        mesh=mesh,
        compiler_params=pltpu.CompilerParams(needs_layout_passes=False,
                                             use_tc_tiling_on_sc=False),
        out_type=[
            jax.ShapeDtypeStruct((_B, _S, 24 * 256), jnp.float32),
            jax.ShapeDtypeStruct((_B, _S, 16), jnp.int32),
        ],
        scratch_types=[
            pltpu.VMEM((3, _N), jnp.float32),
            pltpu.VMEM((3, _SP), jnp.float32),
            pltpu.VMEM((24 * 256,), jnp.float32),
            pltpu.VMEM((16,), jnp.int32),
        ],
    )(pt, nxyz)


def _center_loss(rows_g, cnts):
    acc = jnp.zeros((1, 1), jnp.float32)
    dn_row = (((0,), (0,)), ((), ()))
    for i, (ns, kp, r2, el, scale) in enumerate(_CFGS):
        xr = rows_g[4 * i + 0:4 * i + 1, 0:kp]
        yr = rows_g[4 * i + 1:4 * i + 2, 0:kp]
        zr = rows_g[4 * i + 2:4 * i + 3, 0:kp]
        sqr = rows_g[4 * i + 3:4 * i + 4, 0:kp]
        count = jnp.minimum(cnts[:, i:i + 1], ns)

        kv = jax.lax.broadcasted_iota(jnp.int32, (kp, 1), 0)
        lv = jax.lax.broadcasted_iota(jnp.int32, (1, kp), 1)
        padl = lv >= count
        xb = jnp.where(padl, xr[0:1, 0:1], xr)
        yb = jnp.where(padl, yr[0:1, 0:1], yr)
        zb = jnp.where(padl, zr[0:1, 0:1], zr)
        ggr = jnp.where(padl, sqr[0:1, 0:1], sqr)
        bufT = jnp.concatenate([xb, yb, zb], axis=0).astype(jnp.bfloat16)
        G = jax.lax.dot_general(bufT, bufT, dn_row,
                                preferred_element_type=jnp.float32)
        ggc = jnp.transpose(ggr)
        d2 = jnp.maximum(ggc + ggr - 2.0 * G, 0.0)
        d2 = d2 + jnp.where(lv >= ns, _BIG, 0.0)
        min1 = jnp.min(d2, axis=1, keepdims=True)
        j0 = jnp.min(jnp.where(d2 == min1, lv, kp), axis=1,
                     keepdims=True)
        n2 = jnp.min(d2 + jnp.where(lv == j0, _BIG, 0.0), axis=1,
                     keepdims=True)
        near = jnp.sqrt(jnp.maximum(n2, 1e-12))
        contrib = jnp.where(kv < ns, near + _E, 0.0)
        u = jnp.sum(contrib, axis=0, keepdims=True) / ns
        acc = acc + (u - el) ** 2 / (el + _E) * scale
    return acc


def _group_kernel(rows_ref, cnt_ref, out_ref):
    s0 = pl.program_id(1) * _GRP
    lane_s = jax.lax.broadcasted_iota(jnp.int32, (1, _S), 1)
    row = out_ref[0]
    for g in range(_GRP):
        acc = _center_loss(rows_ref[0, g], cnt_ref[0, g])
        row = jnp.where(lane_s == s0 + g, acc, row)
    out_ref[0] = row


def kernel(pcd):
    pt = jnp.transpose(pcd, (0, 2, 1))
    nxyz = pl.pallas_call(
        _fps_kernel,
        out_shape=jax.ShapeDtypeStruct((_B, 3, _SP), jnp.float32),
    )(pt)
    rows, cnts = _sc_gather(pt, nxyz)
    rows = rows.reshape(_B, _S, 24, 256)
    cnts = cnts.reshape(_B, _S, 1, 16)
    out = pl.pallas_call(
        _group_kernel,
        grid=(_B, _S // _GRP),
        in_specs=[
            pl.BlockSpec((1, _GRP, 24, 256), lambda b, s: (b, s, 0, 0)),
            pl.BlockSpec((1, _GRP, 1, 16), lambda b, s: (b, s, 0, 0)),
        ],
        out_specs=pl.BlockSpec((1, 1, _S), lambda b, s: (b, 0, 0)),
        out_shape=jax.ShapeDtypeStruct((_B, 1, _S), jnp.float32),
        compiler_params=pltpu.CompilerParams(
            dimension_semantics=("parallel", "arbitrary")),
    )(rows, cnts)
    return jnp.mean(out) / len(_PCTS)


# speedup vs baseline: 1.5065x; 1.2961x over previous
import functools
import math

import jax
import jax.numpy as jnp
import numpy as np
from jax.experimental import pallas as pl
from jax.experimental.pallas import tpu as pltpu
from jax.experimental.pallas import tpu_sc as plsc

_RADIUS = 1.0
_PCTS = (0.02, 0.04, 0.06, 0.08, 0.1)
_B, _N, _C = 4, 2048, 3
_S = int(_N * 0.05)
_SP = 128
_GRP = 1
_NW = 32
_GW = (_B * _S + _NW - 1) // _NW
_BIG = np.float32(1e30)
_E = np.float32(0.1)


def _cfg():
    cfgs = []
    for p in _PCTS:
        ns = int(_N * p)
        kp = ((ns + 7) // 8) * 8
        r = math.sqrt(p * _RADIUS)
        r2 = np.float32(r * r)
        disk_area = math.pi * _RADIUS ** 2 / _N
        el = np.float32(math.sqrt(disk_area))
        scale = np.float32((p * 100.0) ** 2)
        cfgs.append((ns, kp, r2, el, scale))
    return tuple(cfgs)


_CFGS = _cfg()


def _fps_kernel(pt_ref, nxyz_ref):
    pt = pt_ref[...]
    lane = jax.lax.broadcasted_iota(jnp.int32, (1, _N), 1)
    lane_s = jax.lax.broadcasted_iota(jnp.int32, (1, 1, _SP), 2)

    def body(i, state):
        dists, far, cacc = state
        oh = lane == far
        cent = jnp.sum(jnp.where(oh[:, None, :], pt, 0.0), axis=2,
                       keepdims=True)
        cacc = jnp.where(lane_s == i, cent, cacc)
        d = jnp.sum((pt - cent) ** 2, axis=1)
        dists = jnp.minimum(dists, d)
        maxv = jnp.max(dists, axis=1, keepdims=True)
        far = jnp.min(jnp.where(dists == maxv, lane, _N), axis=1,
                      keepdims=True)
        return dists, far, cacc

    d0 = jnp.full((_B, _N), 1e10, jnp.float32)
    f0 = jnp.zeros((_B, 1), jnp.int32)
    c0 = jnp.zeros((_B, 3, _SP), jnp.float32)
    _, _, cacc = jax.lax.fori_loop(0, _S, body, (d0, f0, c0))
    nxyz_ref[...] = cacc


def _sc_gather_body(pt_hbm, nxyz_hbm, rows_hbm, cnt_hbm,
                    pt_v, nx_v, row_v, cnt_v):
    wid = jax.lax.axis_index("s") * 2 + jax.lax.axis_index("c")
    il = jax.lax.iota(jnp.int32, 16)

    for i in range(_GW):
        g = wid * _GW + i

        @pl.when(g < _B * _S)
        def _():
            b = g // _S
            s = g - b * _S
            pltpu.sync_copy(pt_hbm.at[b], pt_v)
            pltpu.sync_copy(nxyz_hbm.at[b], nx_v)
            ch = (s // 16) * 16
            lsplat = jnp.full((16, 1), s - ch, jnp.int32)
            gdn = jax.lax.GatherDimensionNumbers(
                offset_dims=(), collapsed_slice_dims=(0,),
                start_index_map=(0,))
            pib = jax.lax.GatherScatterMode.PROMISE_IN_BOUNDS

            def splat(v):
                return jax.lax.gather(v, lsplat, gdn, slice_sizes=(1,),
                                      mode=pib)
            cx = splat(nx_v[0, pl.ds(ch, 16)])
            cy = splat(nx_v[1, pl.ds(ch, 16)])
            cz = splat(nx_v[2, pl.ds(ch, 16)])

            def chunk(j, bases):
                x = pt_v[0, pl.ds(j * 16, 16)]
                y = pt_v[1, pl.ds(j * 16, 16)]
                z = pt_v[2, pl.ds(j * 16, 16)]
                dx, dy, dz = x - cx, y - cy, z - cz
                d2 = dx * dx + dy * dy + dz * dz
                sqv = x * x + y * y + z * z
                out = []
                for pi, (ns, kp, r2, el, scale) in enumerate(_CFGS):
                    m = d2 <= r2
                    rk = jnp.where(m, 1, 0)
                    for k in (1, 2, 4, 8):
                        sh = jax.lax.gather(
                            rk, jnp.maximum(il - k, 0)[:, None], gdn,
                            slice_sizes=(1,), mode=pib)
                        rk = rk + jnp.where(il >= k, sh, 0)
                    pos = bases[pi] + rk - 1
                    wm = m & (pos < ns)
                    r0 = 4 * pi * 256
                    p0 = jnp.where(wm, pos + r0, 20 * 256)
                    plsc.store_scatter(row_v, [p0], x)
                    plsc.store_scatter(row_v, [jnp.where(wm, pos + (r0 + 256),
                                                         20 * 256)], y)
                    plsc.store_scatter(row_v, [jnp.where(wm, pos + (r0 + 512),
                                                         20 * 256)], z)
                    plsc.store_scatter(row_v, [jnp.where(wm, pos + (r0 + 768),
                                                         20 * 256)], sqv)
                    out.append(bases[pi]
                               + plsc.all_reduce_population_count(m))
                return tuple(out)

            z16 = jnp.zeros((16,), jnp.int32)
            bases = jax.lax.fori_loop(0, _N // 16, chunk, (z16,) * 5)
            cnt = jnp.zeros((16,), jnp.int32)
            for pi in range(5):
                cnt = jnp.where(il == pi, bases[pi], cnt)
            cnt_v[...] = cnt
            pltpu.sync_copy(row_v, rows_hbm.at[b, s])
            pltpu.sync_copy(cnt_v, cnt_hbm.at[b, s])


def _sc_gather(pt, nxyz):
    mesh = plsc.VectorSubcoreMesh(core_axis_name="c", subcore_axis_name="s")
    return pl.kernel(
        _sc_gather_body,
        mesh=mesh,
        compiler_params=pltpu.CompilerParams(needs_layout_passes=False,
                                             use_tc_tiling_on_sc=False),
        out_type=[
            jax.ShapeDtypeStruct((_B, _S, 24 * 256), jnp.float32),
            jax.ShapeDtypeStruct((_B, _S, 16), jnp.int32),
        ],
        scratch_types=[
            pltpu.VMEM((3, _N), jnp.float32),
            pltpu.VMEM((3, _SP), jnp.float32),
            pltpu.VMEM((24 * 256,), jnp.float32),
            pltpu.VMEM((16,), jnp.int32),
        ],
    )(pt, nxyz)


def _center_loss(rows_g, cnts):
    acc = jnp.zeros((1, 1), jnp.float32)
    dn_row = (((0,), (0,)), ((), ()))
    for i, (ns, kp, r2, el, scale) in enumerate(_CFGS):
        xr = rows_g[4 * i + 0:4 * i + 1, 0:kp]
        yr = rows_g[4 * i + 1:4 * i + 2, 0:kp]
        zr = rows_g[4 * i + 2:4 * i + 3, 0:kp]
        sqr = rows_g[4 * i + 3:4 * i + 4, 0:kp]
        count = jnp.minimum(cnts[:, i:i + 1], ns)

        kv = jax.lax.broadcasted_iota(jnp.int32, (kp, 1), 0)
        lv = jax.lax.broadcasted_iota(jnp.int32, (1, kp), 1)
        padl = lv >= count
        xb = jnp.where(padl, xr[0:1, 0:1], xr)
        yb = jnp.where(padl, yr[0:1, 0:1], yr)
        zb = jnp.where(padl, zr[0:1, 0:1], zr)
        ggr = jnp.where(padl, sqr[0:1, 0:1], sqr)
        bufT = jnp.concatenate([xb, yb, zb], axis=0).astype(jnp.bfloat16)
        G = jax.lax.dot_general(bufT, bufT, dn_row,
                                preferred_element_type=jnp.float32)
        ggc = jnp.transpose(ggr)
        d2 = jnp.maximum(ggc + ggr - 2.0 * G, 0.0)
        d2 = d2 + jnp.where(lv >= ns, _BIG, 0.0)
        min1 = jnp.min(d2, axis=1, keepdims=True)
        j0 = jnp.min(jnp.where(d2 == min1, lv, kp), axis=1,
                     keepdims=True)
        n2 = jnp.min(d2 + jnp.where(lv == j0, _BIG, 0.0), axis=1,
                     keepdims=True)
        near = jnp.sqrt(jnp.maximum(n2, 1e-12))
        contrib = jnp.where(kv < ns, near + _E, 0.0)
        u = jnp.sum(contrib, axis=0, keepdims=True) / ns
        acc = acc + (u - el) ** 2 / (el + _E) * scale
    return acc


def _group_kernel(rows_ref, cnt_ref, out_ref):
    s0 = pl.program_id(1) * _GRP
    lane_s = jax.lax.broadcasted_iota(jnp.int32, (1, _S), 1)
    row = out_ref[0]
    for g in range(_GRP):
        acc = _center_loss(rows_ref[0, g], cnt_ref[0, g])
        row = jnp.where(lane_s == s0 + g, acc, row)
    out_ref[0] = row


def kernel(pcd):
    pt = jnp.transpose(pcd, (0, 2, 1))
    nxyz = pl.pallas_call(
        _fps_kernel,
        out_shape=jax.ShapeDtypeStruct((_B, 3, _SP), jnp.float32),
    )(pt)
    rows, cnts = _sc_gather(pt, nxyz)
    rows = rows.reshape(_B, _S, 24, 256)
    cnts = cnts.reshape(_B, _S, 1, 16)
    out = pl.pallas_call(
        _group_kernel,
        grid=(_B, _S // _GRP),
        in_specs=[
            pl.BlockSpec((1, _GRP, 24, 256), lambda b, s: (b, s, 0, 0)),
            pl.BlockSpec((1, _GRP, 1, 16), lambda b, s: (b, s, 0, 0)),
        ],
        out_specs=pl.BlockSpec((1, 1, _S), lambda b, s: (b, 0, 0)),
        out_shape=jax.ShapeDtypeStruct((_B, 1, _S), jnp.float32),
        compiler_params=pltpu.CompilerParams(
            dimension_semantics=("parallel", "arbitrary")),
    )(rows, cnts)
    return jnp.mean(out) / len(_PCTS)
